# Initial kernel scaffold; baseline (speedup 1.0000x reference)
import jax
import jax.numpy as jnp
from jax.experimental import pallas as pl


def kernel(x, edge_index, W, b):
    raise NotImplementedError("write your pallas kernel here")


# speedup vs baseline: 7.7755x; 7.7755x over previous
import functools

import jax
import jax.numpy as jnp
from jax import lax
from jax.experimental import pallas as pl
from jax.experimental.pallas import tpu as pltpu
from jax.experimental.pallas import tpu_sc as plsc

N_NODES = 10000
N_PAD = 10240
N_EDGES = 320000
D = 128
K = 80
ZR = 32


def _sc_segment_partials(x, src2d, dst2d):
    info = plsc.get_sparse_core_info()
    nc, ns = info.num_cores, info.num_subcores
    nw = nc * ns
    e_per_w = N_EDGES // nw
    chunks = e_per_w // K
    rows_per_tile = N_PAD // ns

    mesh = plsc.VectorSubcoreMesh(core_axis_name="c", subcore_axis_name="s")

    @functools.partial(
        pl.kernel,
        out_type=jax.ShapeDtypeStruct((nc, N_PAD, D), jnp.float32),
        mesh=mesh,
        scratch_types=[
            pltpu.VMEM((chunks, K), jnp.int32),
            pltpu.VMEM((chunks, K), jnp.int32),
            pltpu.VMEM((K, D), jnp.float32),
            pltpu.VMEM((ZR, D), jnp.float32),
            pltpu.VMEM_SHARED((N_PAD, D), jnp.float32),
            pltpu.SemaphoreType.DMA,
        ],
    )
    def seg_kernel(x_hbm, src_hbm, dst_hbm, out_hbm,
                   sidx, didx, rows, zbuf, acc, sem):
        c = lax.axis_index("c")
        s = lax.axis_index("s")
        wid = s * nc + c

        for r in range(ZR):
            for j in range(D // 16):
                zbuf[r, pl.ds(16 * j, 16)] = jnp.zeros((16,), jnp.float32)
        row_base = s * rows_per_tile

        @pl.loop(0, rows_per_tile // ZR)
        def _zero(k):
            pltpu.sync_copy(zbuf, acc.at[pl.ds(row_base + ZR * k, ZR)])

        plsc.subcore_barrier()

        pltpu.sync_copy(src_hbm.at[wid], sidx)
        pltpu.sync_copy(dst_hbm.at[wid], didx)

        @pl.loop(0, chunks)
        def _edges(ci):
            pltpu.async_copy(x_hbm.at[sidx.at[ci]], rows, sem).wait()
            pltpu.sync_copy(rows, acc.at[didx.at[ci]], add=True)

        plsc.subcore_barrier()
        pltpu.sync_copy(acc.at[pl.ds(row_base, rows_per_tile)],
                        out_hbm.at[c, pl.ds(row_base, rows_per_tile)])

    return seg_kernel(x, src2d, dst2d)


def _tc_apply(partials, W, b):
    nc = partials.shape[0]
    M = partials.shape[1]
    BM = 1024

    def mm(p_ref, w_ref, b_ref, o_ref):
        agg = p_ref[0]
        for i in range(1, nc):
            agg = agg + p_ref[i]
        o_ref[...] = (
            jnp.dot(agg, w_ref[...], preferred_element_type=jnp.float32)
            + b_ref[...]
        )

    return pl.pallas_call(
        mm,
        grid=(M // BM,),
        in_specs=[
            pl.BlockSpec((nc, BM, D), lambda i: (0, i, 0)),
            pl.BlockSpec((D, D), lambda i: (0, 0)),
            pl.BlockSpec((1, D), lambda i: (0, 0)),
        ],
        out_specs=pl.BlockSpec((BM, D), lambda i: (i, 0)),
        out_shape=jax.ShapeDtypeStruct((M, D), jnp.float32),
    )(partials, W, b.reshape(1, D))


def kernel(x, edge_index, W, b):
    nw = 32
    src = edge_index[0].astype(jnp.int32).reshape(nw, N_EDGES // (nw * K), K)
    dst = edge_index[1].astype(jnp.int32).reshape(nw, N_EDGES // (nw * K), K)
    partials = _sc_segment_partials(x, src, dst)
    return _tc_apply(partials, W, b)[:N_NODES]


# speedup vs baseline: 11.5699x; 1.4880x over previous
import functools

import jax
import jax.numpy as jnp
from jax import lax
from jax.experimental import pallas as pl
from jax.experimental.pallas import tpu as pltpu
from jax.experimental.pallas import tpu_sc as plsc

N_NODES = 10000
N_PAD = 10240
N_EDGES = 320000
D = 128
K = 100
G = 20
ZR = 8


def _sc_segment_partials(x, src2d, dst2d):
    info = plsc.get_sparse_core_info()
    nc, ns = info.num_cores, info.num_subcores
    nw = nc * ns
    e_per_w = N_EDGES // nw
    chunks = e_per_w // K
    nsuper = chunks // G
    rows_per_tile = N_PAD // ns

    mesh = plsc.VectorSubcoreMesh(core_axis_name="c", subcore_axis_name="s")

    @functools.partial(
        pl.kernel,
        out_type=jax.ShapeDtypeStruct((nc, N_PAD, D), jnp.float32),
        mesh=mesh,
        scratch_types=[
            pltpu.VMEM((G, K), jnp.int32),
            pltpu.VMEM((G, K), jnp.int32),
            pltpu.VMEM((K, D), jnp.float32),
            pltpu.VMEM((K, D), jnp.float32),
            pltpu.VMEM((ZR, D), jnp.float32),
            pltpu.VMEM_SHARED((N_PAD, D), jnp.float32),
            pltpu.SemaphoreType.DMA,
            pltpu.SemaphoreType.DMA,
        ],
    )
    def seg_kernel(x_hbm, src_hbm, dst_hbm, out_hbm,
                   sidx, didx, rows0, rows1, zbuf, acc, sem0, sem1):
        c = lax.axis_index("c")
        s = lax.axis_index("s")
        wid = s * nc + c

        for r in range(ZR):
            for j in range(D // 16):
                zbuf[r, pl.ds(16 * j, 16)] = jnp.zeros((16,), jnp.float32)
        row_base = s * rows_per_tile

        @pl.loop(0, rows_per_tile // ZR)
        def _zero(k):
            pltpu.sync_copy(zbuf, acc.at[pl.ds(row_base + ZR * k, ZR)])

        plsc.subcore_barrier()

        @pl.loop(0, nsuper)
        def _super(si):
            pltpu.sync_copy(src_hbm.at[wid, si], sidx)
            pltpu.sync_copy(dst_hbm.at[wid, si], didx)
            pltpu.async_copy(x_hbm.at[sidx.at[0]], rows0, sem0)
            pltpu.async_copy(x_hbm.at[sidx.at[1]], rows1, sem1)

            @pl.loop(0, G, step=2)
            def _edges(g):
                pltpu.make_async_copy(x_hbm.at[sidx.at[g]], rows0, sem0).wait()
                pltpu.sync_copy(rows0, acc.at[didx.at[g]], add=True)

                @pl.when(g + 2 < G)
                def _():
                    pltpu.async_copy(x_hbm.at[sidx.at[g + 2]], rows0, sem0)

                pltpu.make_async_copy(x_hbm.at[sidx.at[g + 1]], rows1, sem1).wait()
                pltpu.sync_copy(rows1, acc.at[didx.at[g + 1]], add=True)

                @pl.when(g + 3 < G)
                def _():
                    pltpu.async_copy(x_hbm.at[sidx.at[g + 3]], rows1, sem1)

        plsc.subcore_barrier()
        pltpu.sync_copy(acc.at[pl.ds(row_base, rows_per_tile)],
                        out_hbm.at[c, pl.ds(row_base, rows_per_tile)])

    return seg_kernel(x, src2d, dst2d)


def _tc_apply(partials, W, b):
    nc = partials.shape[0]
    M = partials.shape[1]
    BM = 1024

    def mm(p_ref, w_ref, b_ref, o_ref):
        agg = p_ref[0]
        for i in range(1, nc):
            agg = agg + p_ref[i]
        o_ref[...] = (
            jnp.dot(agg, w_ref[...], preferred_element_type=jnp.float32)
            + b_ref[...]
        )

    return pl.pallas_call(
        mm,
        grid=(M // BM,),
        in_specs=[
            pl.BlockSpec((nc, BM, D), lambda i: (0, i, 0)),
            pl.BlockSpec((D, D), lambda i: (0, 0)),
            pl.BlockSpec((1, D), lambda i: (0, 0)),
        ],
        out_specs=pl.BlockSpec((BM, D), lambda i: (i, 0)),
        out_shape=jax.ShapeDtypeStruct((M, D), jnp.float32),
    )(partials, W, b.reshape(1, D))


def kernel(x, edge_index, W, b):
    nw = 32
    nsuper = N_EDGES // (nw * G * K)
    src = edge_index[0].astype(jnp.int32).reshape(nw, nsuper, G, K)
    dst = edge_index[1].astype(jnp.int32).reshape(nw, nsuper, G, K)
    partials = _sc_segment_partials(x, src, dst)
    return _tc_apply(partials, W, b)[:N_NODES]
